# baseline (speedup 1.0000x reference)
import jax
import jax.numpy as jnp
from jax import lax
from jax.experimental import pallas as pl
from jax.experimental.pallas import tpu as pltpu

M = 16384
D = 2048
BM = 1024


def _mm_body(x_ref, w_ref, m_ref, b_ref, o_ref):
    acc = lax.dot_general(
        x_ref[...], w_ref[...],
        (((1,), (1,)), ((), ())),
        preferred_element_type=jnp.float32,
    )
    o_ref[...] = (acc + b_ref[...]) * m_ref[...]


@jax.jit
def kernel(input, data_mask, W, b):
    B, L, _ = input.shape
    x2 = input.reshape(M, D).astype(jnp.bfloat16)
    maskf = data_mask.reshape(M, 1).astype(jnp.float32)
    wb = W.astype(jnp.bfloat16)
    b2 = b.reshape(1, D)

    out = pl.pallas_call(
        _mm_body,
        grid=(M // BM,),
        in_specs=[
            pl.BlockSpec((BM, D), lambda m: (m, 0)),
            pl.BlockSpec((D, D), lambda m: (0, 0)),
            pl.BlockSpec((BM, 1), lambda m: (m, 0)),
            pl.BlockSpec((1, D), lambda m: (0, 0)),
        ],
        out_specs=pl.BlockSpec((BM, D), lambda m: (m, 0)),
        out_shape=jax.ShapeDtypeStruct((M, D), jnp.float32),
        compiler_params=pltpu.CompilerParams(
            dimension_semantics=("parallel",),
        ),
    )(x2, wb, maskf, b2)
    return out.reshape(B, L, D)


# speedup vs baseline: 1.2507x; 1.2507x over previous
import jax
import jax.numpy as jnp
from jax import lax
from jax.experimental import pallas as pl
from jax.experimental.pallas import tpu as pltpu

M = 16384
D = 2048
BM = 256


def _mm_body(x_ref, w_ref, m_ref, b_ref, o_ref):
    xb = x_ref[...].astype(jnp.bfloat16)
    acc = lax.dot_general(
        xb, w_ref[...],
        (((1,), (1,)), ((), ())),
        preferred_element_type=jnp.float32,
    )
    o_ref[...] = (acc + b_ref[...]) * m_ref[...]


@jax.jit
def kernel(input, data_mask, W, b):
    B, L, _ = input.shape
    x2 = input.reshape(M, D)
    maskf = data_mask.reshape(M, 1).astype(jnp.float32)
    wb = W.astype(jnp.bfloat16)
    b2 = b.reshape(1, D)

    out = pl.pallas_call(
        _mm_body,
        grid=(M // BM,),
        in_specs=[
            pl.BlockSpec((BM, D), lambda m: (m, 0)),
            pl.BlockSpec((D, D), lambda m: (0, 0)),
            pl.BlockSpec((BM, 1), lambda m: (m, 0)),
            pl.BlockSpec((1, D), lambda m: (0, 0)),
        ],
        out_specs=pl.BlockSpec((BM, D), lambda m: (m, 0)),
        out_shape=jax.ShapeDtypeStruct((M, D), jnp.float32),
        compiler_params=pltpu.CompilerParams(
            dimension_semantics=("parallel",),
        ),
    )(x2, wb, maskf, b2)
    return out.reshape(B, L, D)


# speedup vs baseline: 1.3578x; 1.0856x over previous
import jax
import jax.numpy as jnp
from jax import lax
from jax.experimental import pallas as pl
from jax.experimental.pallas import tpu as pltpu

M = 16384
D = 2048
BM = 1024
BN = 512


def _mm_body(x_ref, w_ref, m_ref, b_ref, o_ref):
    xb = x_ref[...].astype(jnp.bfloat16)
    mf = m_ref[...]
    for n0 in range(0, D, BN):
        acc = lax.dot_general(
            xb, w_ref[pl.ds(n0, BN), :],
            (((1,), (1,)), ((), ())),
            preferred_element_type=jnp.float32,
        )
        o_ref[:, pl.ds(n0, BN)] = (acc + b_ref[:, pl.ds(n0, BN)]) * mf


@jax.jit
def kernel(input, data_mask, W, b):
    B, L, _ = input.shape
    x2 = input.reshape(M, D)
    maskf = data_mask.reshape(M, 1).astype(jnp.float32)
    wb = W.astype(jnp.bfloat16)
    b2 = b.reshape(1, D)

    out = pl.pallas_call(
        _mm_body,
        grid=(M // BM,),
        in_specs=[
            pl.BlockSpec((BM, D), lambda m: (m, 0)),
            pl.BlockSpec((D, D), lambda m: (0, 0)),
            pl.BlockSpec((BM, 1), lambda m: (m, 0)),
            pl.BlockSpec((1, D), lambda m: (0, 0)),
        ],
        out_specs=pl.BlockSpec((BM, D), lambda m: (m, 0)),
        out_shape=jax.ShapeDtypeStruct((M, D), jnp.float32),
        compiler_params=pltpu.CompilerParams(
            dimension_semantics=("parallel",),
        ),
    )(x2, wb, maskf, b2)
    return out.reshape(B, L, D)
